# baseline (speedup 1.0000x reference)
import functools
import math

import jax
import jax.numpy as jnp
from jax import lax
from jax.experimental import pallas as pl
from jax.experimental.pallas import tpu as pltpu
from jax.experimental.pallas import tpu_sc as plsc

_VOCAB = 1000000
_EMB = 64
_B = 4096
_L = 200
_BT = _B * _L

_info = plsc.get_sparse_core_info()
_NC = _info.num_cores
_NS = _info.num_subcores
_NW = _NC * _NS
_CHUNK = 128
_PER_W = _BT // _NW
_CPW = _PER_W // _CHUNK

_SCALE = math.sqrt(_EMB)


def _body(tok_hbm, table_hbm, out_hbm, idx_v, rows_v, sem):
    wid = lax.axis_index("s") * _NC + lax.axis_index("c")
    pltpu.sync_copy(tok_hbm.at[pl.ds(wid * _CPW, _CPW)], idx_v)
    base = wid * _PER_W

    def chunk(j, carry):
        pltpu.async_copy(table_hbm.at[idx_v.at[j]], rows_v, sem).wait()

        def scale_rows(r, c2):
            for seg in range(4):
                sl = pl.ds(seg * 16, 16)
                rows_v[r, sl] = rows_v[r, sl] * _SCALE
            return c2

        lax.fori_loop(0, _CHUNK, scale_rows, 0, unroll=4)
        pltpu.sync_copy(rows_v, out_hbm.at[pl.ds(base + j * _CHUNK, _CHUNK)])
        return carry

    lax.fori_loop(0, _CPW, chunk, 0)


_mesh = plsc.VectorSubcoreMesh(core_axis_name="c", subcore_axis_name="s")

_gather = functools.partial(
    pl.kernel,
    mesh=_mesh,
    out_type=jax.ShapeDtypeStruct((_BT, _EMB), jnp.float32),
    scratch_types=[
        pltpu.VMEM((_CPW, _CHUNK), jnp.int32),
        pltpu.VMEM((_CHUNK, _EMB), jnp.float32),
        pltpu.SemaphoreType.DMA,
    ],
    compiler_params=pltpu.CompilerParams(use_tc_tiling_on_sc=False),
)(_body)


def kernel(tokens, table):
    tok = tokens.reshape(-1).astype(jnp.int32).reshape(_NW * _CPW, _CHUNK)
    out = _gather(tok, table)
    return out.reshape(_B, _L, _EMB)


# speedup vs baseline: 1.1659x; 1.1659x over previous
import functools
import math

import jax
import jax.numpy as jnp
from jax import lax
from jax.experimental import pallas as pl
from jax.experimental.pallas import tpu as pltpu
from jax.experimental.pallas import tpu_sc as plsc

_VOCAB = 1000000
_EMB = 64
_B = 4096
_L = 200
_BT = _B * _L

_info = plsc.get_sparse_core_info()
_NC = _info.num_cores
_NS = _info.num_subcores
_NW = _NC * _NS
_CHUNK = 128
_PER_W = _BT // _NW
_CPW = _PER_W // _CHUNK
_NBUF = 4

_SCALE = math.sqrt(_EMB)


def _body(tok_hbm, table_hbm, out_hbm, idx_v, rows, gsem, osem):
    wid = lax.axis_index("s") * _NC + lax.axis_index("c")
    pltpu.sync_copy(tok_hbm.at[pl.ds(wid * _CPW, _CPW)], idx_v)
    base = wid * _PER_W

    def gstart(j, b):
        pltpu.async_copy(table_hbm.at[idx_v.at[j]], rows[b], gsem[b])

    def gwait(j, b):
        pltpu.make_async_copy(table_hbm.at[idx_v.at[j]], rows[b], gsem[b]).wait()

    def ostart(j, b):
        pltpu.async_copy(rows[b], out_hbm.at[pl.ds(base + j * _CHUNK, _CHUNK)], osem[b])

    def owait(j, b):
        pltpu.make_async_copy(
            rows[b], out_hbm.at[pl.ds(base + j * _CHUNK, _CHUNK)], osem[b]
        ).wait()

    def scale(b):
        @pl.loop(0, _CHUNK, unroll=8)
        def _(r):
            for seg in range(4):
                sl = pl.ds(seg * 16, 16)
                rows[b][r, sl] = rows[b][r, sl] * _SCALE

    gstart(0, 0)
    gstart(1, 1)

    for b in range(_NBUF):
        j = b
        if j >= 2:
            owait(j - 2, (b - 2) % _NBUF)
        gstart(j + 2, (b + 2) % _NBUF)
        gwait(j, b)
        scale(b)
        ostart(j, b)

    @pl.loop(0, (_CPW - 2 * _NBUF) // _NBUF)
    def _(r):
        j0 = _NBUF + r * _NBUF
        for b in range(_NBUF):
            j = j0 + b
            owait(j - 2, (b - 2) % _NBUF)
            gstart(j + 2, (b + 2) % _NBUF)
            gwait(j, b)
            scale(b)
            ostart(j, b)

    for b in range(_NBUF):
        j = _CPW - _NBUF + b
        owait(j - 2, (b - 2) % _NBUF)
        if j + 2 < _CPW:
            gstart(j + 2, (b + 2) % _NBUF)
        gwait(j, b)
        scale(b)
        ostart(j, b)

    owait(_CPW - 2, (_NBUF - 2) % _NBUF)
    owait(_CPW - 1, _NBUF - 1)


_mesh = plsc.VectorSubcoreMesh(core_axis_name="c", subcore_axis_name="s")

_gather = functools.partial(
    pl.kernel,
    mesh=_mesh,
    out_type=jax.ShapeDtypeStruct((_BT, _EMB), jnp.float32),
    scratch_types=[
        pltpu.VMEM((_CPW, _CHUNK), jnp.int32),
        [pltpu.VMEM((_CHUNK, _EMB), jnp.float32) for _ in range(_NBUF)],
        [pltpu.SemaphoreType.DMA for _ in range(_NBUF)],
        [pltpu.SemaphoreType.DMA for _ in range(_NBUF)],
    ],
    compiler_params=pltpu.CompilerParams(use_tc_tiling_on_sc=False),
)(_body)


def kernel(tokens, table):
    tok = tokens.reshape(-1).astype(jnp.int32).reshape(_NW * _CPW, _CHUNK)
    out = _gather(tok, table)
    return out.reshape(_B, _L, _EMB)
